# baseline (speedup 1.0000x reference)
import jax
import jax.numpy as jnp
from jax import lax
from jax.experimental import pallas as pl
from jax.experimental.pallas import tpu as pltpu

_BF = 64
_BB = 32
_SPLIT = 512


def _bcast_kernel(dummy_ref, pos_ref, tab_hbm, out_hbm, stage, sem):
    del dummy_ref
    c, _, s = pos_ref.shape
    rows = out_hbm.shape[0]
    b = rows // c
    rest = b - _SPLIT
    n = rest // _BF
    for p in range(c):
        row8 = pos_ref[p]
        for k in range(_BF // 8):
            stage[pl.ds(p * _BF + 8 * k, 8), :] = row8
    for p in range(c):
        for i in range(n):
            pltpu.make_async_copy(
                stage.at[pl.ds(p * _BF, _BF), :],
                out_hbm.at[pl.ds(p * b + _SPLIT + i * _BF, _BF), :],
                sem,
            ).start()

    def _inner(src_ref, out_ref):
        for k in range(_BB // 8):
            out_ref[pl.ds(8 * k, 8), :] = src_ref[...]

    nblk = b // _BB
    pltpu.emit_pipeline(
        _inner,
        grid=(c, _SPLIT // _BB),
        in_specs=[pl.BlockSpec((8, s), lambda p, i: (p, 0))],
        out_specs=[pl.BlockSpec((_BB, s), lambda p, i: (p * nblk + i, 0))],
    )(tab_hbm, out_hbm)

    for p in range(c):
        for i in range(n):
            pltpu.make_async_copy(
                stage.at[pl.ds(p * _BF, _BF), :],
                out_hbm.at[pl.ds(p * b + _SPLIT + i * _BF, _BF), :],
                sem,
            ).wait()


def kernel(dummy, position_ids):
    b = dummy.shape[0]
    _, c, one, s = position_ids.shape
    table = lax.bitcast_convert_type(position_ids.reshape(c, 1, s), jnp.bfloat16)
    table8 = jnp.broadcast_to(table, (c, 8, s))
    idx2d = dummy.reshape(1, b)
    out = pl.pallas_call(
        _bcast_kernel,
        in_specs=[
            pl.BlockSpec((1, b), lambda: (0, 0)),
            pl.BlockSpec((c, 8, s), lambda: (0, 0, 0)),
            pl.BlockSpec(memory_space=pl.ANY),
        ],
        out_specs=pl.BlockSpec(memory_space=pl.ANY),
        out_shape=jax.ShapeDtypeStruct((c * b, s), jnp.bfloat16),
        scratch_shapes=[
            pltpu.VMEM((c * _BF, s), jnp.bfloat16),
            pltpu.SemaphoreType.DMA,
        ],
    )(idx2d, table8, table8.reshape(c * 8, s))
    out16 = lax.bitcast_convert_type(out, position_ids.dtype)
    out3 = out16.reshape(c, b, s)
    return jnp.transpose(out3, (1, 0, 2)).reshape(b, c, one, s)
